# Initial kernel scaffold; baseline (speedup 1.0000x reference)
import jax
import jax.numpy as jnp
from jax.experimental import pallas as pl


def kernel(x, edge_index, batch, W1, b1, g1, bt1, pw1, W2, b2, g2, bt2, pw2, Wf, bf, Wo, bo):
    raise NotImplementedError("write your pallas kernel here")


# speedup vs baseline: 6.1062x; 6.1062x over previous
import functools

import jax
import jax.numpy as jnp
from jax import lax
from jax.experimental import pallas as pl
from jax.experimental.pallas import tpu as pltpu
from jax.experimental.pallas import tpu_sc as plsc

N = 10000
E = 160000
F_IN = 256
H = 1024
OUT = 128
K1 = 5000
K2 = 2500

NC = 2
NS = 16
LN = 128
NPAD = 10240
RPS = NPAD // NS
ECH = 80
EPAD = NS * ECH * LN

BM = 400
GRID = N // BM

_f32 = jnp.float32
_i32 = jnp.int32

@functools.lru_cache(maxsize=None)
def _mesh():
    return plsc.VectorSubcoreMesh(
        core_axis_name="c", subcore_axis_name="s",
        num_cores=NC, num_subcores=NS)


def _sc_deg_body(src_hbm, dst_hbm, vtab_hbm, zeros_hbm, out_hbm,
                 idx_s, idx_d, vals_v, accum, sem):
    c = lax.axis_index("c")
    s = lax.axis_index("s")
    half = ECH // NC
    pltpu.sync_copy(zeros_hbm, accum.at[pl.ds(s * RPS, RPS)])
    pltpu.sync_copy(src_hbm.at[s, pl.ds(c * half, half)], idx_s)
    pltpu.sync_copy(dst_hbm.at[s, pl.ds(c * half, half)], idx_d)
    plsc.subcore_barrier()

    def step(j, carry):
        pltpu.async_copy(vtab_hbm.at[idx_s.at[j]], vals_v, sem).wait()
        pltpu.sync_copy(vals_v, accum.at[idx_d.at[j]], add=True)
        return carry

    lax.fori_loop(0, half, step, 0)
    plsc.subcore_barrier()
    pltpu.sync_copy(accum.at[pl.ds(s * RPS, RPS)],
                    out_hbm.at[c, pl.ds(s * RPS, RPS)])


@functools.lru_cache(maxsize=None)
def _sc_deg_kernel():
    return pl.kernel(
        _sc_deg_body,
        out_type=jax.ShapeDtypeStruct((NC, NPAD), _f32),
        mesh=_mesh(),
        scratch_types=[
            pltpu.VMEM((ECH // NC, LN), _i32),
            pltpu.VMEM((ECH // NC, LN), _i32),
            pltpu.VMEM((LN,), _f32),
            pltpu.VMEM_SHARED((NPAD,), _f32),
            pltpu.SemaphoreType.DMA,
        ],
    )


def _sc_deg(src_l, dst_l, vtab, zeros1):
    return _sc_deg_kernel()(src_l, dst_l, vtab, zeros1)


def _sc_agg_body(nk, srck_hbm, dst_hbm, tab_hbm, zeros_hbm, out_hbm,
                 idx_s, idx_d, rows_v, accum, sem):
    c = lax.axis_index("c")
    s = lax.axis_index("s")
    pltpu.sync_copy(dst_hbm.at[s], idx_d)
    for p in range(nk // NC):
        k = p * NC + c
        pltpu.sync_copy(zeros_hbm, accum.at[pl.ds(s * RPS, RPS)])
        pltpu.sync_copy(srck_hbm.at[k, s], idx_s)
        plsc.subcore_barrier()

        def step(j, carry):
            pltpu.async_copy(tab_hbm.at[idx_s.at[j]], rows_v, sem).wait()
            pltpu.sync_copy(rows_v, accum.at[idx_d.at[j]], add=True)
            return carry

        lax.fori_loop(0, ECH, step, 0)
        plsc.subcore_barrier()
        pltpu.sync_copy(accum.at[pl.ds(s * RPS, RPS)],
                        out_hbm.at[pl.ds(s * RPS, RPS), pl.ds(k * LN, LN)])
        plsc.subcore_barrier()


@functools.lru_cache(maxsize=None)
def _sc_agg_kernel(nk):
    return pl.kernel(
        functools.partial(_sc_agg_body, nk),
        out_type=jax.ShapeDtypeStruct((NPAD, nk * LN), _f32),
        mesh=_mesh(),
        scratch_types=[
            pltpu.VMEM((ECH, LN), _i32),
            pltpu.VMEM((ECH, LN), _i32),
            pltpu.VMEM((LN, LN), _f32),
            pltpu.VMEM_SHARED((NPAD, LN), _f32),
            pltpu.SemaphoreType.DMA,
        ],
    )


def _sc_agg(nk, srck, dst_l, tab, zeros2):
    return _sc_agg_kernel(nk)(srck, dst_l, tab, zeros2)


def _scale_rows_body(da_ref, db_ref, x_ref, xs_ref, di_ref, ds_ref):
    d = da_ref[...] + db_ref[...] + 1.0
    di = lax.rsqrt(d)
    di_ref[...] = di
    ds_ref[...] = di * di
    xs_ref[...] = x_ref[...] * di


@functools.lru_cache(maxsize=None)
def _scale_rows_kernel(K):
    return pl.pallas_call(
        _scale_rows_body,
        grid=(GRID,),
        in_specs=[
            pl.BlockSpec((BM, 1), lambda i: (i, 0)),
            pl.BlockSpec((BM, 1), lambda i: (i, 0)),
            pl.BlockSpec((BM, K), lambda i: (i, 0)),
        ],
        out_specs=[
            pl.BlockSpec((BM, K), lambda i: (i, 0)),
            pl.BlockSpec((BM, 1), lambda i: (i, 0)),
            pl.BlockSpec((BM, 1), lambda i: (i, 0)),
        ],
        out_shape=[
            jax.ShapeDtypeStruct((N, K), _f32),
            jax.ShapeDtypeStruct((N, 1), _f32),
            jax.ShapeDtypeStruct((N, 1), _f32),
        ],
    )


def _mm_body(u_ref, su_ref, v_ref, sv_ref, m_ref, w_ref, b_ref, y_ref, st_ref):
    i = pl.program_id(0)
    m = m_ref[...]
    p = (u_ref[...] * su_ref[...] + v_ref[...] * sv_ref[...]) * m
    y = jnp.dot(p, w_ref[...], preferred_element_type=_f32,
                precision=lax.Precision.HIGHEST) + b_ref[...]
    y_ref[...] = y
    ym = y * m

    @pl.when(i == 0)
    def _():
        st_ref[...] = jnp.zeros_like(st_ref)

    st_ref[0:1, :] += jnp.sum(ym, axis=0, keepdims=True)


@functools.lru_cache(maxsize=None)
def _mm_kernel(K):
    return pl.pallas_call(
        _mm_body,
        grid=(GRID,),
        in_specs=[
            pl.BlockSpec((BM, K), lambda i: (i, 0)),
            pl.BlockSpec((BM, 1), lambda i: (i, 0)),
            pl.BlockSpec((BM, K), lambda i: (i, 0)),
            pl.BlockSpec((BM, 1), lambda i: (i, 0)),
            pl.BlockSpec((BM, 1), lambda i: (i, 0)),
            pl.BlockSpec((K, H), lambda i: (0, 0)),
            pl.BlockSpec((1, H), lambda i: (0, 0)),
        ],
        out_specs=[
            pl.BlockSpec((BM, H), lambda i: (i, 0)),
            pl.BlockSpec((8, H), lambda i: (0, 0)),
        ],
        out_shape=[
            jax.ShapeDtypeStruct((N, H), _f32),
            jax.ShapeDtypeStruct((8, H), _f32),
        ],
    )


def _bns_body(cnt, y_ref, st_ref, g_ref, bt_ref, pw_ref, m_ref,
              h_ref, sc_ref, sv_ref):
    p = pl.program_id(0)
    i = pl.program_id(1)
    mu = st_ref[0:1, :] * (1.0 / cnt)

    @pl.when((p == 0) & (i == 0))
    def _():
        sv_ref[...] = jnp.zeros_like(sv_ref)

    @pl.when(p == 0)
    def _():
        d = (y_ref[...] - mu) * m_ref[...]
        sv_ref[0:1, :] += jnp.sum(d * d, axis=0, keepdims=True)

    @pl.when(p == 1)
    def _():
        var = sv_ref[0:1, :] * (1.0 / cnt)
        rstd = lax.rsqrt(var + 1e-5)
        h = jnp.maximum(
            (y_ref[...] - mu) * rstd * g_ref[...] + bt_ref[...], 0.0)
        h_ref[...] = h
        pw = pw_ref[...]
        pwn = pw * lax.rsqrt(jnp.sum(pw * pw))
        sc_ref[...] = jnp.dot(h, pwn.reshape(H, 1), preferred_element_type=_f32,
                              precision=lax.Precision.HIGHEST)


@functools.lru_cache(maxsize=None)
def _bns_kernel(cnt):
    return pl.pallas_call(
        functools.partial(_bns_body, float(cnt)),
        grid=(2, GRID),
        in_specs=[
            pl.BlockSpec((BM, H), lambda p, i: (i, 0)),
            pl.BlockSpec((8, H), lambda p, i: (0, 0)),
            pl.BlockSpec((1, H), lambda p, i: (0, 0)),
            pl.BlockSpec((1, H), lambda p, i: (0, 0)),
            pl.BlockSpec((1, H), lambda p, i: (0, 0)),
            pl.BlockSpec((BM, 1), lambda p, i: (i, 0)),
        ],
        out_specs=[
            pl.BlockSpec((BM, H), lambda p, i: (i, 0)),
            pl.BlockSpec((BM, 1), lambda p, i: (i, 0)),
        ],
        out_shape=[
            jax.ShapeDtypeStruct((N, H), _f32),
            jax.ShapeDtypeStruct((N, 1), _f32),
        ],
        scratch_shapes=[pltpu.VMEM((8, H), _f32)],
    )


def _topk_body(kk, sc_ref, el_ref, sel_ref, tf_ref):
    sc = jnp.where(el_ref[...] > 0, sc_ref[...], -jnp.inf)
    bi = lax.bitcast_convert_type(sc, _i32)
    uk = jnp.where(bi < 0, ~bi, bi ^ jnp.int32(-2147483648)).astype(jnp.uint32)

    def rbody(t, pfx):
        bit = lax.shift_right_logical(
            jnp.uint32(2147483648), t.astype(jnp.uint32))
        cand = pfx | bit
        cnt = jnp.sum((uk >= cand).astype(_f32))
        return jnp.where(cnt >= kk, cand, pfx)

    vk = lax.fori_loop(0, 32, rbody, jnp.uint32(0))
    gt = uk > vk
    tie = uk == vk
    n_gt = jnp.sum(gt.astype(_f32))
    need = kk - n_gt
    tf = tie.astype(_f32)
    r0 = lax.broadcasted_iota(_i32, (LN, LN), 0)
    r1 = lax.broadcasted_iota(_i32, (LN, LN), 1)
    m128 = (r0 < r1).astype(_f32)
    q0 = lax.broadcasted_iota(_i32, (ECH, ECH), 0)
    q1 = lax.broadcasted_iota(_i32, (ECH, ECH), 1)
    m80t = (q1 < q0).astype(_f32)
    excl = jnp.dot(tf, m128, preferred_element_type=_f32)
    rowtot = jnp.sum(tf, axis=1, keepdims=True)
    rowexcl = jnp.dot(m80t, rowtot, preferred_element_type=_f32)
    rank = rowexcl + excl
    sel = jnp.logical_or(gt, jnp.logical_and(tie, rank < need)).astype(_f32)
    sel_ref[...] = sel
    tf_ref[...] = sel * jnp.tanh(sc)


@functools.lru_cache(maxsize=None)
def _topk_kernel(kk):
    return pl.pallas_call(
        functools.partial(_topk_body, float(kk)),
        in_specs=[
            pl.BlockSpec((ECH, LN), lambda: (0, 0)),
            pl.BlockSpec((ECH, LN), lambda: (0, 0)),
        ],
        out_specs=[
            pl.BlockSpec((ECH, LN), lambda: (0, 0)),
            pl.BlockSpec((ECH, LN), lambda: (0, 0)),
        ],
        out_shape=[
            jax.ShapeDtypeStruct((ECH, LN), _f32),
            jax.ShapeDtypeStruct((ECH, LN), _f32),
        ],
    )


def _colsum_body(emit, h_ref, t_ref, *out_refs):
    i = pl.program_id(0)
    hs = h_ref[...] * t_ref[...]
    if emit:
        out_refs[0][...] = hs
    xs_ref = out_refs[-1]

    @pl.when(i == 0)
    def _():
        xs_ref[...] = jnp.zeros_like(xs_ref)

    xs_ref[0:1, :] += jnp.sum(hs, axis=0, keepdims=True)


@functools.lru_cache(maxsize=None)
def _colsum_kernel(emit):
    outs = ([pl.BlockSpec((BM, H), lambda i: (i, 0))] if emit else [])
    outs.append(pl.BlockSpec((8, H), lambda i: (0, 0)))
    shapes = ([jax.ShapeDtypeStruct((N, H), _f32)] if emit else [])
    shapes.append(jax.ShapeDtypeStruct((8, H), _f32))
    return pl.pallas_call(
        functools.partial(_colsum_body, emit),
        grid=(GRID,),
        in_specs=[
            pl.BlockSpec((BM, H), lambda i: (i, 0)),
            pl.BlockSpec((BM, 1), lambda i: (i, 0)),
        ],
        out_specs=outs,
        out_shape=shapes,
    )


def _final_body(x1_ref, x2_ref, wf_ref, bf_ref, wo_ref, bo_ref, o_ref):
    z = x1_ref[0:1, :] * (1.0 / K1) + x2_ref[0:1, :] * (1.0 / K2)
    a = jnp.maximum(
        jnp.dot(z, wf_ref[...], preferred_element_type=_f32) + bf_ref[...], 0.0)
    o_ref[...] = jnp.dot(a, wo_ref[...], preferred_element_type=_f32) + bo_ref[...]


@functools.lru_cache(maxsize=None)
def _final_kernel():
    return pl.pallas_call(
        _final_body,
        in_specs=[
            pl.BlockSpec((8, H), lambda: (0, 0)),
            pl.BlockSpec((8, H), lambda: (0, 0)),
            pl.BlockSpec((H, 512), lambda: (0, 0)),
            pl.BlockSpec((1, 512), lambda: (0, 0)),
            pl.BlockSpec((512, OUT), lambda: (0, 0)),
            pl.BlockSpec((1, OUT), lambda: (0, 0)),
        ],
        out_specs=pl.BlockSpec((1, OUT), lambda: (0, 0)),
        out_shape=jax.ShapeDtypeStruct((1, OUT), _f32),
    )


def kernel(x, edge_index, batch, W1, b1, g1, bt1, pw1,
           W2, b2, g2, bt2, pw2, Wf, bf, Wo, bo):
    src = edge_index[0]
    dst = edge_index[1]
    pad = EPAD - E
    src_l = jnp.concatenate(
        [src, jnp.full((pad,), N, _i32)]).reshape(NS, ECH, LN)
    dst_l = jnp.concatenate(
        [dst, jnp.full((pad,), N, _i32)]).reshape(NS, ECH, LN)
    src2k = src_l[None] * 2 + jnp.arange(2, dtype=_i32)[:, None, None, None]
    src8k = src_l[None] * 8 + jnp.arange(8, dtype=_i32)[:, None, None, None]
    zeros1 = jnp.zeros((RPS,), _f32)
    zeros2 = jnp.zeros((RPS, LN), _f32)
    ones_col = jnp.ones((N, 1), _f32)

    deg1p = _sc_deg(src_l, dst_l, jnp.ones((NPAD,), _f32), zeros1)
    xs, dinv1, dinvsq1 = _scale_rows_kernel(F_IN)(
        deg1p[0, :N, None], deg1p[1, :N, None], x)
    tab1 = jnp.pad(xs, ((0, 1), (0, 0))).reshape((N + 1) * 2, LN)
    agg1 = _sc_agg(2, src2k, dst_l, tab1, zeros2)[:N]
    h1pre, st1 = _mm_kernel(F_IN)(
        agg1, dinv1, x, dinvsq1, ones_col, W1, b1[None])
    h1, sc1 = _bns_kernel(N)(h1pre, st1, g1[None], bt1[None], pw1[None],
                             ones_col)

    scp1 = jnp.pad(sc1[:, 0], (0, NPAD - N)).reshape(ECH, LN)
    el1 = jnp.pad(jnp.ones((N,), _f32), (0, NPAD - N)).reshape(ECH, LN)
    sel1g, tf1g = _topk_kernel(K1)(scp1, el1)
    sel1 = sel1g.reshape(-1)[:N]
    tf1 = tf1g.reshape(-1)[:N]
    h1s, x1sum = _colsum_kernel(True)(h1, tf1[:, None])

    deg2p = _sc_deg(src_l, dst_l, jnp.pad(sel1, (0, NPAD - N)), zeros1)
    hs2, dinv2, dinvsq2 = _scale_rows_kernel(H)(
        deg2p[0, :N, None], deg2p[1, :N, None], h1s)
    tab2 = jnp.pad(hs2, ((0, 1), (0, 0))).reshape((N + 1) * 8, LN)
    agg2 = _sc_agg(8, src8k, dst_l, tab2, zeros2)[:N]
    h2pre, st2 = _mm_kernel(H)(
        agg2, dinv2, h1s, dinvsq2, sel1[:, None], W2, b2[None])
    h2, sc2 = _bns_kernel(K1)(h2pre, st2, g2[None], bt2[None], pw2[None],
                              sel1[:, None])

    scp2 = jnp.pad(sc2[:, 0], (0, NPAD - N)).reshape(ECH, LN)
    el2 = jnp.pad(sel1, (0, NPAD - N)).reshape(ECH, LN)
    _, tf2g = _topk_kernel(K2)(scp2, el2)
    tf2 = tf2g.reshape(-1)[:N]
    (x2sum,) = _colsum_kernel(False)(h2, tf2[:, None])

    return _final_kernel()(x1sum, x2sum, Wf, bf[None], Wo, bo[None])


# speedup vs baseline: 6.6398x; 1.0874x over previous
import functools

import jax
import jax.numpy as jnp
from jax import lax
from jax.experimental import pallas as pl
from jax.experimental.pallas import tpu as pltpu
from jax.experimental.pallas import tpu_sc as plsc

N = 10000
E = 160000
F_IN = 256
H = 1024
OUT = 128
K1 = 5000
K2 = 2500

NC = 2
NS = 16
LN = 128
NPAD = 10240
RPS = NPAD // NS
CB = 64
NCH = 160
EPAD = NS * NCH * CB
TKR = 80
TKC = 128

BM = 400
GRID = N // BM

_f32 = jnp.float32
_i32 = jnp.int32

@functools.lru_cache(maxsize=None)
def _mesh():
    return plsc.VectorSubcoreMesh(
        core_axis_name="c", subcore_axis_name="s",
        num_cores=NC, num_subcores=NS)


def _sc_deg_body(src_hbm, dst_hbm, vtab_hbm, zeros_hbm, out_hbm,
                 idx_s, idx_d, vals_v, accum, sem):
    c = lax.axis_index("c")
    s = lax.axis_index("s")
    half = NCH // NC
    pltpu.sync_copy(zeros_hbm, accum.at[pl.ds(s * RPS, RPS)])
    pltpu.sync_copy(src_hbm.at[s, pl.ds(c * half, half)], idx_s)
    pltpu.sync_copy(dst_hbm.at[s, pl.ds(c * half, half)], idx_d)
    plsc.subcore_barrier()

    def step(j, carry):
        pltpu.async_copy(vtab_hbm.at[idx_s.at[j]], vals_v, sem).wait()
        pltpu.sync_copy(vals_v, accum.at[idx_d.at[j]], add=True)
        return carry

    lax.fori_loop(0, half, step, 0)
    plsc.subcore_barrier()
    pltpu.sync_copy(accum.at[pl.ds(s * RPS, RPS)],
                    out_hbm.at[c, pl.ds(s * RPS, RPS)])


@functools.lru_cache(maxsize=None)
def _sc_deg_kernel():
    return pl.kernel(
        _sc_deg_body,
        out_type=jax.ShapeDtypeStruct((NC, NPAD), _f32),
        mesh=_mesh(),
        scratch_types=[
            pltpu.VMEM((NCH // NC, CB), _i32),
            pltpu.VMEM((NCH // NC, CB), _i32),
            pltpu.VMEM((CB,), _f32),
            pltpu.VMEM_SHARED((NPAD,), _f32),
            pltpu.SemaphoreType.DMA,
        ],
    )


def _sc_deg(src_l, dst_l, vtab, zeros1):
    return _sc_deg_kernel()(src_l, dst_l, vtab, zeros1)


def _sc_agg_body(nk, srck_hbm, dst_hbm, tab_hbm, zeros_hbm, out_hbm,
                 idx_s, idx_d, rows_a, rows_b, accum, sema, semb):
    c = lax.axis_index("c")
    s = lax.axis_index("s")
    hch = NCH // 2
    pltpu.sync_copy(dst_hbm.at[s], idx_d)
    for p in range(nk // NC):
        k = p * NC + c
        pltpu.sync_copy(zeros_hbm, accum.at[pl.ds(s * RPS, RPS)])
        plsc.subcore_barrier()
        for hh in range(2):
            pltpu.sync_copy(srck_hbm.at[k, s, pl.ds(hh * hch, hch)], idx_s)
            pltpu.async_copy(tab_hbm.at[idx_s.at[0]], rows_a, sema)

            def step(jj, carry, hh=hh):
                j0 = 2 * jj
                d0 = hh * hch + j0
                pltpu.async_copy(tab_hbm.at[idx_s.at[j0 + 1]], rows_b, semb)
                pltpu.make_async_copy(
                    tab_hbm.at[idx_s.at[j0]], rows_a, sema).wait()
                pltpu.sync_copy(rows_a, accum.at[idx_d.at[d0]], add=True)

                @pl.when(jj < hch // 2 - 1)
                def _():
                    pltpu.async_copy(
                        tab_hbm.at[idx_s.at[j0 + 2]], rows_a, sema)

                pltpu.make_async_copy(
                    tab_hbm.at[idx_s.at[j0 + 1]], rows_b, semb).wait()
                pltpu.sync_copy(rows_b, accum.at[idx_d.at[d0 + 1]], add=True)
                return carry

            lax.fori_loop(0, hch // 2, step, 0)
        plsc.subcore_barrier()
        pltpu.sync_copy(accum.at[pl.ds(s * RPS, RPS)],
                        out_hbm.at[pl.ds(s * RPS, RPS), pl.ds(k * LN, LN)])
        plsc.subcore_barrier()


@functools.lru_cache(maxsize=None)
def _sc_agg_kernel(nk):
    return pl.kernel(
        functools.partial(_sc_agg_body, nk),
        out_type=jax.ShapeDtypeStruct((NPAD, nk * LN), _f32),
        mesh=_mesh(),
        scratch_types=[
            pltpu.VMEM((NCH // 2, CB), _i32),
            pltpu.VMEM((NCH, CB), _i32),
            pltpu.VMEM((CB, LN), _f32),
            pltpu.VMEM((CB, LN), _f32),
            pltpu.VMEM_SHARED((NPAD, LN), _f32),
            pltpu.SemaphoreType.DMA,
            pltpu.SemaphoreType.DMA,
        ],
    )


def _sc_agg(nk, srck, dst_l, tab, zeros2):
    return _sc_agg_kernel(nk)(srck, dst_l, tab, zeros2)


def _scale_rows_body(da_ref, db_ref, x_ref, xs_ref, di_ref, ds_ref):
    d = da_ref[...] + db_ref[...] + 1.0
    di = lax.rsqrt(d)
    di_ref[...] = di
    ds_ref[...] = di * di
    xs_ref[...] = x_ref[...] * di


@functools.lru_cache(maxsize=None)
def _scale_rows_kernel(K):
    return pl.pallas_call(
        _scale_rows_body,
        grid=(GRID,),
        in_specs=[
            pl.BlockSpec((BM, 1), lambda i: (i, 0)),
            pl.BlockSpec((BM, 1), lambda i: (i, 0)),
            pl.BlockSpec((BM, K), lambda i: (i, 0)),
        ],
        out_specs=[
            pl.BlockSpec((BM, K), lambda i: (i, 0)),
            pl.BlockSpec((BM, 1), lambda i: (i, 0)),
            pl.BlockSpec((BM, 1), lambda i: (i, 0)),
        ],
        out_shape=[
            jax.ShapeDtypeStruct((N, K), _f32),
            jax.ShapeDtypeStruct((N, 1), _f32),
            jax.ShapeDtypeStruct((N, 1), _f32),
        ],
    )


def _mm_body(u_ref, su_ref, v_ref, sv_ref, m_ref, w_ref, b_ref, y_ref, st_ref):
    i = pl.program_id(0)
    m = m_ref[...]
    p = (u_ref[...] * su_ref[...] + v_ref[...] * sv_ref[...]) * m
    y = jnp.dot(p, w_ref[...], preferred_element_type=_f32,
                precision=lax.Precision.HIGHEST) + b_ref[...]
    y_ref[...] = y
    ym = y * m

    @pl.when(i == 0)
    def _():
        st_ref[...] = jnp.zeros_like(st_ref)

    st_ref[0:1, :] += jnp.sum(ym, axis=0, keepdims=True)


@functools.lru_cache(maxsize=None)
def _mm_kernel(K):
    return pl.pallas_call(
        _mm_body,
        grid=(GRID,),
        in_specs=[
            pl.BlockSpec((BM, K), lambda i: (i, 0)),
            pl.BlockSpec((BM, 1), lambda i: (i, 0)),
            pl.BlockSpec((BM, K), lambda i: (i, 0)),
            pl.BlockSpec((BM, 1), lambda i: (i, 0)),
            pl.BlockSpec((BM, 1), lambda i: (i, 0)),
            pl.BlockSpec((K, H), lambda i: (0, 0)),
            pl.BlockSpec((1, H), lambda i: (0, 0)),
        ],
        out_specs=[
            pl.BlockSpec((BM, H), lambda i: (i, 0)),
            pl.BlockSpec((8, H), lambda i: (0, 0)),
        ],
        out_shape=[
            jax.ShapeDtypeStruct((N, H), _f32),
            jax.ShapeDtypeStruct((8, H), _f32),
        ],
    )


def _bns_body(cnt, y_ref, st_ref, g_ref, bt_ref, pw_ref, m_ref,
              h_ref, sc_ref, sv_ref):
    p = pl.program_id(0)
    i = pl.program_id(1)
    mu = st_ref[0:1, :] * (1.0 / cnt)

    @pl.when((p == 0) & (i == 0))
    def _():
        sv_ref[...] = jnp.zeros_like(sv_ref)

    @pl.when(p == 0)
    def _():
        d = (y_ref[...] - mu) * m_ref[...]
        sv_ref[0:1, :] += jnp.sum(d * d, axis=0, keepdims=True)

    @pl.when(p == 1)
    def _():
        var = sv_ref[0:1, :] * (1.0 / cnt)
        rstd = lax.rsqrt(var + 1e-5)
        h = jnp.maximum(
            (y_ref[...] - mu) * rstd * g_ref[...] + bt_ref[...], 0.0)
        h_ref[...] = h
        pw = pw_ref[...]
        pwn = pw * lax.rsqrt(jnp.sum(pw * pw))
        sc_ref[...] = jnp.dot(h, pwn.reshape(H, 1), preferred_element_type=_f32,
                              precision=lax.Precision.HIGHEST)


@functools.lru_cache(maxsize=None)
def _bns_kernel(cnt):
    return pl.pallas_call(
        functools.partial(_bns_body, float(cnt)),
        grid=(2, GRID),
        in_specs=[
            pl.BlockSpec((BM, H), lambda p, i: (i, 0)),
            pl.BlockSpec((8, H), lambda p, i: (0, 0)),
            pl.BlockSpec((1, H), lambda p, i: (0, 0)),
            pl.BlockSpec((1, H), lambda p, i: (0, 0)),
            pl.BlockSpec((1, H), lambda p, i: (0, 0)),
            pl.BlockSpec((BM, 1), lambda p, i: (i, 0)),
        ],
        out_specs=[
            pl.BlockSpec((BM, H), lambda p, i: (i, 0)),
            pl.BlockSpec((BM, 1), lambda p, i: (i, 0)),
        ],
        out_shape=[
            jax.ShapeDtypeStruct((N, H), _f32),
            jax.ShapeDtypeStruct((N, 1), _f32),
        ],
        scratch_shapes=[pltpu.VMEM((8, H), _f32)],
    )


def _topk_body(kk, sc_ref, el_ref, sel_ref, tf_ref):
    sc = jnp.where(el_ref[...] > 0, sc_ref[...], -jnp.inf)
    bi = lax.bitcast_convert_type(sc, _i32)
    uk = jnp.where(bi < 0, ~bi, bi ^ jnp.int32(-2147483648)).astype(jnp.uint32)

    def rbody(t, pfx):
        bit = lax.shift_right_logical(
            jnp.uint32(2147483648), t.astype(jnp.uint32))
        cand = pfx | bit
        cnt = jnp.sum((uk >= cand).astype(_f32))
        return jnp.where(cnt >= kk, cand, pfx)

    vk = lax.fori_loop(0, 32, rbody, jnp.uint32(0))
    gt = uk > vk
    tie = uk == vk
    n_gt = jnp.sum(gt.astype(_f32))
    need = kk - n_gt
    tf = tie.astype(_f32)
    r0 = lax.broadcasted_iota(_i32, (LN, LN), 0)
    r1 = lax.broadcasted_iota(_i32, (LN, LN), 1)
    m128 = (r0 < r1).astype(_f32)
    q0 = lax.broadcasted_iota(_i32, (TKR, TKR), 0)
    q1 = lax.broadcasted_iota(_i32, (TKR, TKR), 1)
    m80t = (q1 < q0).astype(_f32)
    excl = jnp.dot(tf, m128, preferred_element_type=_f32)
    rowtot = jnp.sum(tf, axis=1, keepdims=True)
    rowexcl = jnp.dot(m80t, rowtot, preferred_element_type=_f32)
    rank = rowexcl + excl
    sel = jnp.logical_or(gt, jnp.logical_and(tie, rank < need)).astype(_f32)
    sel_ref[...] = sel
    tf_ref[...] = sel * jnp.tanh(sc)


@functools.lru_cache(maxsize=None)
def _topk_kernel(kk):
    return pl.pallas_call(
        functools.partial(_topk_body, float(kk)),
        in_specs=[
            pl.BlockSpec((TKR, TKC), lambda: (0, 0)),
            pl.BlockSpec((TKR, TKC), lambda: (0, 0)),
        ],
        out_specs=[
            pl.BlockSpec((TKR, TKC), lambda: (0, 0)),
            pl.BlockSpec((TKR, TKC), lambda: (0, 0)),
        ],
        out_shape=[
            jax.ShapeDtypeStruct((TKR, TKC), _f32),
            jax.ShapeDtypeStruct((TKR, TKC), _f32),
        ],
    )


def _colsum_body(emit, h_ref, t_ref, *out_refs):
    i = pl.program_id(0)
    hs = h_ref[...] * t_ref[...]
    if emit:
        out_refs[0][...] = hs
    xs_ref = out_refs[-1]

    @pl.when(i == 0)
    def _():
        xs_ref[...] = jnp.zeros_like(xs_ref)

    xs_ref[0:1, :] += jnp.sum(hs, axis=0, keepdims=True)


@functools.lru_cache(maxsize=None)
def _colsum_kernel(emit):
    outs = ([pl.BlockSpec((BM, H), lambda i: (i, 0))] if emit else [])
    outs.append(pl.BlockSpec((8, H), lambda i: (0, 0)))
    shapes = ([jax.ShapeDtypeStruct((N, H), _f32)] if emit else [])
    shapes.append(jax.ShapeDtypeStruct((8, H), _f32))
    return pl.pallas_call(
        functools.partial(_colsum_body, emit),
        grid=(GRID,),
        in_specs=[
            pl.BlockSpec((BM, H), lambda i: (i, 0)),
            pl.BlockSpec((BM, 1), lambda i: (i, 0)),
        ],
        out_specs=outs,
        out_shape=shapes,
    )


def _final_body(x1_ref, x2_ref, wf_ref, bf_ref, wo_ref, bo_ref, o_ref):
    z = x1_ref[0:1, :] * (1.0 / K1) + x2_ref[0:1, :] * (1.0 / K2)
    a = jnp.maximum(
        jnp.dot(z, wf_ref[...], preferred_element_type=_f32) + bf_ref[...], 0.0)
    o_ref[...] = jnp.dot(a, wo_ref[...], preferred_element_type=_f32) + bo_ref[...]


@functools.lru_cache(maxsize=None)
def _final_kernel():
    return pl.pallas_call(
        _final_body,
        in_specs=[
            pl.BlockSpec((8, H), lambda: (0, 0)),
            pl.BlockSpec((8, H), lambda: (0, 0)),
            pl.BlockSpec((H, 512), lambda: (0, 0)),
            pl.BlockSpec((1, 512), lambda: (0, 0)),
            pl.BlockSpec((512, OUT), lambda: (0, 0)),
            pl.BlockSpec((1, OUT), lambda: (0, 0)),
        ],
        out_specs=pl.BlockSpec((1, OUT), lambda: (0, 0)),
        out_shape=jax.ShapeDtypeStruct((1, OUT), _f32),
    )


def kernel(x, edge_index, batch, W1, b1, g1, bt1, pw1,
           W2, b2, g2, bt2, pw2, Wf, bf, Wo, bo):
    src = edge_index[0]
    dst = edge_index[1]
    pad = EPAD - E
    src_l = jnp.concatenate(
        [src, jnp.full((pad,), N, _i32)]).reshape(NS, NCH, CB)
    dst_l = jnp.concatenate(
        [dst, jnp.full((pad,), N, _i32)]).reshape(NS, NCH, CB)
    src2k = src_l[None] * 2 + jnp.arange(2, dtype=_i32)[:, None, None, None]
    src8k = src_l[None] * 8 + jnp.arange(8, dtype=_i32)[:, None, None, None]
    zeros1 = jnp.zeros((RPS,), _f32)
    zeros2 = jnp.zeros((RPS, LN), _f32)
    ones_col = jnp.ones((N, 1), _f32)

    deg1p = _sc_deg(src_l, dst_l, jnp.ones((NPAD,), _f32), zeros1)
    xs, dinv1, dinvsq1 = _scale_rows_kernel(F_IN)(
        deg1p[0, :N, None], deg1p[1, :N, None], x)
    tab1 = jnp.pad(xs, ((0, 1), (0, 0))).reshape((N + 1) * 2, LN)
    agg1 = _sc_agg(2, src2k, dst_l, tab1, zeros2)[:N]
    h1pre, st1 = _mm_kernel(F_IN)(
        agg1, dinv1, x, dinvsq1, ones_col, W1, b1[None])
    h1, sc1 = _bns_kernel(N)(h1pre, st1, g1[None], bt1[None], pw1[None],
                             ones_col)

    scp1 = jnp.pad(sc1[:, 0], (0, NPAD - N)).reshape(TKR, TKC)
    el1 = jnp.pad(jnp.ones((N,), _f32), (0, NPAD - N)).reshape(TKR, TKC)
    sel1g, tf1g = _topk_kernel(K1)(scp1, el1)
    sel1 = sel1g.reshape(-1)[:N]
    tf1 = tf1g.reshape(-1)[:N]
    h1s, x1sum = _colsum_kernel(True)(h1, tf1[:, None])

    deg2p = _sc_deg(src_l, dst_l, jnp.pad(sel1, (0, NPAD - N)), zeros1)
    hs2, dinv2, dinvsq2 = _scale_rows_kernel(H)(
        deg2p[0, :N, None], deg2p[1, :N, None], h1s)
    tab2 = jnp.pad(hs2, ((0, 1), (0, 0))).reshape((N + 1) * 8, LN)
    agg2 = _sc_agg(8, src8k, dst_l, tab2, zeros2)[:N]
    h2pre, st2 = _mm_kernel(H)(
        agg2, dinv2, h1s, dinvsq2, sel1[:, None], W2, b2[None])
    h2, sc2 = _bns_kernel(K1)(h2pre, st2, g2[None], bt2[None], pw2[None],
                              sel1[:, None])

    scp2 = jnp.pad(sc2[:, 0], (0, NPAD - N)).reshape(TKR, TKC)
    el2 = jnp.pad(sel1, (0, NPAD - N)).reshape(TKR, TKC)
    _, tf2g = _topk_kernel(K2)(scp2, el2)
    tf2 = tf2g.reshape(-1)[:N]
    (x2sum,) = _colsum_kernel(False)(h2, tf2[:, None])

    return _final_kernel()(x1sum, x2sum, Wf, bf[None], Wo, bo[None])
